# baseline (speedup 1.0000x reference)
import jax
import jax.numpy as jnp
from jax.experimental import pallas as pl
from jax.experimental.pallas import tpu as pltpu

_ALPHA = 1.702
_LIMIT = 7.0
_GU_SPLIT = 8
_D_SPLIT = 8
_BUFS = 3


def _outer_body(ri_ref, rw_ref, x_ref, wgu_hbm, bgu_ref, wd_hbm, bd_ref,
                out_ref, sel_even_ref, sel_odd_ref):
    G, D = _GU_SPLIT, _D_SPLIT
    E = bgu_ref.shape[0]
    i2 = sel_even_ref.shape[0]
    i = i2 // 2
    hg = wgu_hbm.shape[2]
    idd = wd_hbm.shape[2]

    out_ref[...] = jnp.zeros_like(out_ref)
    row = jax.lax.broadcasted_iota(jnp.int32, (i2, i), 0)
    col = jax.lax.broadcasted_iota(jnp.int32, (i2, i), 1)
    sel_even_ref[...] = (row == 2 * col).astype(jnp.float32)
    sel_odd_ref[...] = (row == 2 * col + 1).astype(jnp.float32)
    x = x_ref[...]

    def inner(indices, *wrefs):
        e = indices[0]
        wgu_refs = wrefs[:G]
        wd_refs = wrefs[G:G + D]
        gu = bgu_ref[e]
        for q, wref in enumerate(wgu_refs):
            gu = gu + jnp.dot(x[:, q * hg:(q + 1) * hg], wref[0, 0],
                              preferred_element_type=jnp.float32)
        gate_full = jnp.minimum(gu, _LIMIT)
        up_full = jnp.clip(gu, -_LIMIT, _LIMIT)
        glu_full = gate_full * jax.nn.sigmoid(gate_full * _ALPHA)
        glu = jnp.dot(glu_full, sel_even_ref[...],
                      preferred_element_type=jnp.float32)
        up = jnp.dot(up_full, sel_odd_ref[...],
                     preferred_element_type=jnp.float32)
        gated = (up + 1.0) * glu
        out = bd_ref[e]
        for q, wref in enumerate(wd_refs):
            out = out + jnp.dot(gated[:, q * idd:(q + 1) * idd], wref[0, 0],
                                preferred_element_type=jnp.float32)
        w = jnp.sum(rw_ref[...] * (ri_ref[...] == e).astype(jnp.float32),
                    axis=1, keepdims=True)
        out_ref[...] += out * w

    def gu_spec(q):
        return pl.BlockSpec((1, 1, hg, i2), lambda e, q=q: (e, q, 0, 0),
                            pipeline_mode=pl.Buffered(buffer_count=_BUFS))

    def d_spec(q):
        return pl.BlockSpec((1, 1, idd, bd_ref.shape[1]),
                            lambda e, q=q: (e, q, 0, 0),
                            pipeline_mode=pl.Buffered(buffer_count=_BUFS))

    pipe = pltpu.emit_pipeline(
        inner,
        grid=(E,),
        in_specs=[gu_spec(q) for q in range(G)] + [d_spec(q) for q in range(D)],
        _explicit_indices=True,
    )
    pipe(*([wgu_hbm] * G), *([wd_hbm] * D))


def kernel(hidden_states, router_indices, routing_weights, gate_up_proj,
           gate_up_proj_bias, down_proj, down_proj_bias):
    T, H = hidden_states.shape
    E, _, I2 = gate_up_proj.shape
    I = I2 // 2
    G, D = _GU_SPLIT, _D_SPLIT

    wgu4 = gate_up_proj.reshape(E, G, H // G, I2)
    wd4 = down_proj.reshape(E, D, I // D, H)

    out = pl.pallas_call(
        _outer_body,
        in_specs=[
            pl.BlockSpec(memory_space=pltpu.VMEM),
            pl.BlockSpec(memory_space=pltpu.VMEM),
            pl.BlockSpec(memory_space=pltpu.VMEM),
            pl.BlockSpec(memory_space=pl.ANY),
            pl.BlockSpec(memory_space=pltpu.VMEM),
            pl.BlockSpec(memory_space=pl.ANY),
            pl.BlockSpec(memory_space=pltpu.VMEM),
        ],
        out_specs=pl.BlockSpec(memory_space=pltpu.VMEM),
        out_shape=jax.ShapeDtypeStruct((T, H), hidden_states.dtype),
        scratch_shapes=[
            pltpu.VMEM((I2, I), jnp.float32),
            pltpu.VMEM((I2, I), jnp.float32),
        ],
    )(router_indices, routing_weights, hidden_states, wgu4,
      gate_up_proj_bias, wd4, down_proj_bias)
    return out
